# baseline (speedup 1.0000x reference)
import math

import jax
import jax.numpy as jnp
from jax.experimental import pallas as pl
from jax.experimental.pallas import tpu as pltpu

_NCORES = 2


def _tchunk(T):
    return math.gcd(T, 8)


def _fused_kernel(x_hbm, wih_hbm, whh_hbm, b_hbm, fcw_hbm, fcb_hbm,
                  r_hbm, h0_hbm, h1_hbm, h2_hbm, h3_hbm,
                  xs_scr, gx_scr, rout_scr, head_scr, h_scr, c_scr,
                  wih_scr, whh_scr, b_scr, fcw_scr, fcb_scr,
                  in_sems, w_sems, rout_sem, head_sem):
    T, Bc, I = xs_scr.shape
    Hp = h_scr.shape[1]
    FCp = fcb_scr.shape[1]
    C = h0_hbm.shape[2]
    j = pl.program_id(0)
    b0 = j * Bc
    TC = _tchunk(T)
    nchunks = T // TC
    head_hbm = (h0_hbm, h1_hbm, h2_hbm, h3_hbm)

    def x_copy(t):
        return pltpu.make_async_copy(
            x_hbm.at[pl.ds(b0, Bc), t, :], xs_scr.at[t],
            in_sems.at[t // TC])

    in_copies = [x_copy(t) for t in range(T)]
    w_copies = [
        pltpu.make_async_copy(src, dst, w_sems.at[i])
        for i, (src, dst) in enumerate([
            (wih_hbm, wih_scr), (b_hbm, b_scr), (whh_hbm, whh_scr),
            (fcw_hbm, fcw_scr), (fcb_hbm, fcb_scr)])
    ]
    for t in range(TC):
        in_copies[t].start()
    for cp in w_copies:
        cp.start()
    for t in range(TC, T):
        in_copies[t].start()

    h_scr[...] = jnp.zeros_like(h_scr)
    c_scr[...] = jnp.zeros_like(c_scr)

    out_copies = []

    def project_chunk(ck):
        if ck == 0:
            w_copies[0].wait()
            w_copies[1].wait()
        for t in range(ck * TC, (ck + 1) * TC):
            in_copies[t].wait()
        rows = ck * TC * Bc
        gx_scr[rows:rows + TC * Bc, :] = (
            jnp.dot(
                xs_scr[ck * TC:(ck + 1) * TC].reshape(TC * Bc, I),
                wih_scr[...], preferred_element_type=jnp.float32)
            + b_scr[...]
        )

    def recur_step(t):
        if t == 0:
            w_copies[2].wait()
        gates = gx_scr[t * Bc:(t + 1) * Bc, :] + jnp.dot(
            h_scr[...], whh_scr[...], preferred_element_type=jnp.float32)
        i_g = jax.nn.sigmoid(gates[:, 0 * Hp:1 * Hp])
        f_g = jax.nn.sigmoid(gates[:, 1 * Hp:2 * Hp])
        g_g = jnp.tanh(gates[:, 2 * Hp:3 * Hp])
        o_g = jax.nn.sigmoid(gates[:, 3 * Hp:4 * Hp])
        c_new = f_g * c_scr[...] + i_g * g_g
        h_new = o_g * jnp.tanh(c_new)
        c_scr[...] = c_new
        h_scr[...] = h_new
        rout_scr[t] = h_new
        cp = pltpu.make_async_copy(
            rout_scr.at[t], r_hbm.at[pl.ds(b0, Bc), t, :], rout_sem.at[0])
        cp.start()
        out_copies.append(cp)

    def head_chunk(ck):
        if ck == 0:
            w_copies[3].wait()
            w_copies[4].wait()
        head_scr[ck * TC:(ck + 1) * TC] = (
            jnp.dot(
                rout_scr[ck * TC:(ck + 1) * TC].reshape(TC * Bc, Hp),
                fcw_scr[...], preferred_element_type=jnp.float32)
            + fcb_scr[...]
        ).reshape(TC, Bc, FCp)
        for t in range(ck * TC, (ck + 1) * TC):
            for f in range(FCp // C):
                cp = pltpu.make_async_copy(
                    head_scr.at[t, :, pl.ds(f * C, C)],
                    head_hbm[f].at[pl.ds(b0, Bc), t, :],
                    head_sem.at[0])
                cp.start()
                out_copies.append(cp)

    project_chunk(0)
    if nchunks > 1:
        project_chunk(1)
    for ck in range(nchunks):
        for t in range(ck * TC, (ck + 1) * TC):
            recur_step(t)
        if ck + 2 < nchunks:
            project_chunk(ck + 2)
        if ck >= 1:
            head_chunk(ck - 1)
    head_chunk(nchunks - 1)

    for cp in out_copies:
        cp.wait()


def kernel(x, w_ih_t, w_hh_t, b, fc_w, fc_b):
    B, T, I = x.shape
    Hp4 = w_ih_t.shape[1]
    Hp = Hp4 // 4
    FCp = fc_w.shape[1]
    F, C = 4, 128
    Bc = B // _NCORES
    TC = _tchunk(T)

    outs = pl.pallas_call(
        _fused_kernel,
        out_shape=(
            jax.ShapeDtypeStruct((B, T, Hp), jnp.float32),
        ) + tuple(
            jax.ShapeDtypeStruct((B, T, C), jnp.float32) for _ in range(F)
        ),
        grid=(_NCORES,),
        in_specs=[
            pl.BlockSpec(memory_space=pl.ANY) for _ in range(6)
        ],
        out_specs=tuple(
            pl.BlockSpec(memory_space=pl.ANY) for _ in range(1 + F)),
        scratch_shapes=[
            pltpu.VMEM((T, Bc, I), jnp.float32),
            pltpu.VMEM((T * Bc, Hp4), jnp.float32),
            pltpu.VMEM((T, Bc, Hp), jnp.float32),
            pltpu.VMEM((T, Bc, FCp), jnp.float32),
            pltpu.VMEM((Bc, Hp), jnp.float32),
            pltpu.VMEM((Bc, Hp), jnp.float32),
            pltpu.VMEM((I, Hp4), jnp.float32),
            pltpu.VMEM((Hp, Hp4), jnp.float32),
            pltpu.VMEM((1, Hp4), jnp.float32),
            pltpu.VMEM((Hp, FCp), jnp.float32),
            pltpu.VMEM((1, FCp), jnp.float32),
            pltpu.SemaphoreType.DMA((T // TC,)),
            pltpu.SemaphoreType.DMA((5,)),
            pltpu.SemaphoreType.DMA((1,)),
            pltpu.SemaphoreType.DMA((1,)),
        ],
        compiler_params=pltpu.CompilerParams(
            dimension_semantics=("parallel",)),
    )(x, w_ih_t, w_hh_t, b, fc_w, fc_b)

    r_out = outs[0]
    model_out = list(outs[1:])
    return model_out, r_out


# speedup vs baseline: 1.0028x; 1.0028x over previous
import math

import jax
import jax.numpy as jnp
from jax.experimental import pallas as pl
from jax.experimental.pallas import tpu as pltpu

_NCORES = 2


def _tchunk(T):
    return math.gcd(T, 8)


def _fused_kernel(x_hbm, wih_ref, whh_ref, b_ref, fcw_ref, fcb_ref,
                  r_hbm, h0_hbm, h1_hbm, h2_hbm, h3_hbm,
                  xs_scr, gx_scr, rout_scr, head_scr, h_scr, c_scr,
                  in_sems, rout_sem, head_sem):
    T, Bc, I = xs_scr.shape
    Hp = h_scr.shape[1]
    FCp = fcb_ref.shape[1]
    C = h0_hbm.shape[2]
    j = pl.program_id(0)
    b0 = j * Bc
    TC = _tchunk(T)
    nchunks = T // TC
    head_hbm = (h0_hbm, h1_hbm, h2_hbm, h3_hbm)

    in_copies = []
    for t in range(T):
        cp = pltpu.make_async_copy(
            x_hbm.at[pl.ds(b0, Bc), t, :], xs_scr.at[t],
            in_sems.at[t // TC])
        cp.start()
        in_copies.append(cp)

    h_scr[...] = jnp.zeros_like(h_scr)
    c_scr[...] = jnp.zeros_like(c_scr)

    out_copies = []

    def project_chunk(ck):
        for t in range(ck * TC, (ck + 1) * TC):
            in_copies[t].wait()
        rows = ck * TC * Bc
        gx_scr[rows:rows + TC * Bc, :] = (
            jnp.dot(
                xs_scr[ck * TC:(ck + 1) * TC].reshape(TC * Bc, I),
                wih_ref[...], preferred_element_type=jnp.float32)
            + b_ref[...]
        )

    def recur_step(t):
        gates = gx_scr[t * Bc:(t + 1) * Bc, :] + jnp.dot(
            h_scr[...], whh_ref[...], preferred_element_type=jnp.float32)
        i_g = jax.nn.sigmoid(gates[:, 0 * Hp:1 * Hp])
        f_g = jax.nn.sigmoid(gates[:, 1 * Hp:2 * Hp])
        g_g = jnp.tanh(gates[:, 2 * Hp:3 * Hp])
        o_g = jax.nn.sigmoid(gates[:, 3 * Hp:4 * Hp])
        c_new = f_g * c_scr[...] + i_g * g_g
        h_new = o_g * jnp.tanh(c_new)
        c_scr[...] = c_new
        h_scr[...] = h_new
        rout_scr[t] = h_new
        cp = pltpu.make_async_copy(
            rout_scr.at[t], r_hbm.at[pl.ds(b0, Bc), t, :], rout_sem.at[0])
        cp.start()
        out_copies.append(cp)

    def head_chunk(ck):
        head_scr[ck * TC:(ck + 1) * TC] = (
            jnp.dot(
                rout_scr[ck * TC:(ck + 1) * TC].reshape(TC * Bc, Hp),
                fcw_ref[...], preferred_element_type=jnp.float32)
            + fcb_ref[...]
        ).reshape(TC, Bc, FCp)
        for t in range(ck * TC, (ck + 1) * TC):
            for f in range(FCp // C):
                cp = pltpu.make_async_copy(
                    head_scr.at[t, :, pl.ds(f * C, C)],
                    head_hbm[f].at[pl.ds(b0, Bc), t, :],
                    head_sem.at[0])
                cp.start()
                out_copies.append(cp)

    project_chunk(0)
    if nchunks > 1:
        project_chunk(1)
    for ck in range(nchunks):
        for t in range(ck * TC, (ck + 1) * TC):
            recur_step(t)
        if ck + 2 < nchunks:
            project_chunk(ck + 2)
        if ck >= 1:
            head_chunk(ck - 1)
    head_chunk(nchunks - 1)

    for cp in out_copies:
        cp.wait()


def kernel(x, w_ih_t, w_hh_t, b, fc_w, fc_b):
    B, T, I = x.shape
    Hp4 = w_ih_t.shape[1]
    Hp = Hp4 // 4
    FCp = fc_w.shape[1]
    F, C = 4, 128
    Bc = B // _NCORES
    TC = _tchunk(T)

    outs = pl.pallas_call(
        _fused_kernel,
        out_shape=(
            jax.ShapeDtypeStruct((B, T, Hp), jnp.float32),
        ) + tuple(
            jax.ShapeDtypeStruct((B, T, C), jnp.float32) for _ in range(F)
        ),
        grid=(_NCORES,),
        in_specs=[
            pl.BlockSpec(memory_space=pl.ANY),
            pl.BlockSpec((I, Hp4), lambda j: (0, 0)),
            pl.BlockSpec((Hp, Hp4), lambda j: (0, 0)),
            pl.BlockSpec((1, Hp4), lambda j: (0, 0)),
            pl.BlockSpec((Hp, FCp), lambda j: (0, 0)),
            pl.BlockSpec((1, FCp), lambda j: (0, 0)),
        ],
        out_specs=tuple(
            pl.BlockSpec(memory_space=pl.ANY) for _ in range(1 + F)),
        scratch_shapes=[
            pltpu.VMEM((T, Bc, I), jnp.float32),
            pltpu.VMEM((T * Bc, Hp4), jnp.float32),
            pltpu.VMEM((T, Bc, Hp), jnp.float32),
            pltpu.VMEM((T, Bc, FCp), jnp.float32),
            pltpu.VMEM((Bc, Hp), jnp.float32),
            pltpu.VMEM((Bc, Hp), jnp.float32),
            pltpu.SemaphoreType.DMA((T // TC,)),
            pltpu.SemaphoreType.DMA((1,)),
            pltpu.SemaphoreType.DMA((1,)),
        ],
        compiler_params=pltpu.CompilerParams(
            dimension_semantics=("parallel",)),
    )(x, w_ih_t, w_hh_t, b, fc_w, fc_b)

    r_out = outs[0]
    model_out = list(outs[1:])
    return model_out, r_out


# speedup vs baseline: 1.1530x; 1.1498x over previous
import math

import jax
import jax.numpy as jnp
from jax.experimental import pallas as pl
from jax.experimental.pallas import tpu as pltpu

_NCORES = 2


def _tchunk(T):
    return math.gcd(T, 8)


def _fused_kernel(x_hbm, wih_ref, whh_ref, b_ref, fcw_ref, fcb_ref,
                  r_hbm, h0_hbm, h1_hbm, h2_hbm, h3_hbm,
                  xs_scr, gx_scr, rout_scr, head_scr, h_scr, c_scr,
                  in_sems, rout_sem, head_sem):
    T, Bc, I = xs_scr.shape
    Hp = h_scr.shape[1]
    FCp = fcb_ref.shape[1]
    C = h0_hbm.shape[2]
    j = pl.program_id(0)
    b0 = j * Bc
    TC = _tchunk(T)
    nchunks = T // TC
    head_hbm = (h0_hbm, h1_hbm, h2_hbm, h3_hbm)

    in_copies = []
    for t in range(T):
        cp = pltpu.make_async_copy(
            x_hbm.at[pl.ds(b0, Bc), t, :], xs_scr.at[t],
            in_sems.at[t // TC])
        cp.start()
        in_copies.append(cp)

    h_scr[...] = jnp.zeros_like(h_scr)
    c_scr[...] = jnp.zeros_like(c_scr)

    out_copies = []
    whh = whh_ref[...]
    fcw = fcw_ref[...]

    def project_chunk(ck):
        for t in range(ck * TC, (ck + 1) * TC):
            in_copies[t].wait()
        rows = ck * TC * Bc
        gx_scr[rows:rows + TC * Bc, :] = (
            jnp.dot(
                xs_scr[ck * TC:(ck + 1) * TC].reshape(TC * Bc, I),
                wih_ref[...], preferred_element_type=jnp.float32)
            + b_ref[...]
        )

    def recur_step(t):
        gates = gx_scr[t * Bc:(t + 1) * Bc, :] + jnp.dot(
            h_scr[...], whh, preferred_element_type=jnp.float32)
        i_g = jax.nn.sigmoid(gates[:, 0 * Hp:1 * Hp])
        f_g = jax.nn.sigmoid(gates[:, 1 * Hp:2 * Hp])
        g_g = jnp.tanh(gates[:, 2 * Hp:3 * Hp])
        o_g = jax.nn.sigmoid(gates[:, 3 * Hp:4 * Hp])
        c_new = f_g * c_scr[...] + i_g * g_g
        h_new = o_g * jnp.tanh(c_new)
        c_scr[...] = c_new
        h_scr[...] = h_new
        rout_scr[t] = h_new
        cp = pltpu.make_async_copy(
            rout_scr.at[t], r_hbm.at[pl.ds(b0, Bc), t, :], rout_sem.at[0])
        cp.start()
        out_copies.append(cp)

    def head_chunk(ck):
        head_scr[ck * TC:(ck + 1) * TC] = (
            jnp.dot(
                rout_scr[ck * TC:(ck + 1) * TC].reshape(TC * Bc, Hp),
                fcw, preferred_element_type=jnp.float32)
            + fcb_ref[...]
        ).reshape(TC, Bc, FCp)
        for t in range(ck * TC, (ck + 1) * TC):
            for f in range(FCp // C):
                cp = pltpu.make_async_copy(
                    head_scr.at[t, :, pl.ds(f * C, C)],
                    head_hbm[f].at[pl.ds(b0, Bc), t, :],
                    head_sem.at[0])
                cp.start()
                out_copies.append(cp)

    project_chunk(0)
    if nchunks > 1:
        project_chunk(1)
    for ck in range(nchunks):
        for t in range(ck * TC, (ck + 1) * TC):
            recur_step(t)
        if ck + 2 < nchunks:
            project_chunk(ck + 2)
        if ck >= 1:
            head_chunk(ck - 1)
    head_chunk(nchunks - 1)

    for cp in out_copies:
        cp.wait()


def kernel(x, w_ih_t, w_hh_t, b, fc_w, fc_b):
    B, T, I = x.shape
    Hp4 = w_ih_t.shape[1]
    Hp = Hp4 // 4
    FCp = fc_w.shape[1]
    F, C = 4, 128
    Bc = B // _NCORES
    TC = _tchunk(T)

    outs = pl.pallas_call(
        _fused_kernel,
        out_shape=(
            jax.ShapeDtypeStruct((B, T, Hp), jnp.float32),
        ) + tuple(
            jax.ShapeDtypeStruct((B, T, C), jnp.float32) for _ in range(F)
        ),
        grid=(_NCORES,),
        in_specs=[
            pl.BlockSpec(memory_space=pl.ANY),
            pl.BlockSpec((I, Hp4), lambda j: (0, 0)),
            pl.BlockSpec((Hp, Hp4), lambda j: (0, 0)),
            pl.BlockSpec((1, Hp4), lambda j: (0, 0)),
            pl.BlockSpec((Hp, FCp), lambda j: (0, 0)),
            pl.BlockSpec((1, FCp), lambda j: (0, 0)),
        ],
        out_specs=tuple(
            pl.BlockSpec(memory_space=pl.ANY) for _ in range(1 + F)),
        scratch_shapes=[
            pltpu.VMEM((T, Bc, I), jnp.float32),
            pltpu.VMEM((T * Bc, Hp4), jnp.float32),
            pltpu.VMEM((T, Bc, Hp), jnp.float32),
            pltpu.VMEM((T, Bc, FCp), jnp.float32),
            pltpu.VMEM((Bc, Hp), jnp.float32),
            pltpu.VMEM((Bc, Hp), jnp.float32),
            pltpu.SemaphoreType.DMA((T // TC,)),
            pltpu.SemaphoreType.DMA((1,)),
            pltpu.SemaphoreType.DMA((1,)),
        ],
        compiler_params=pltpu.CompilerParams(
            dimension_semantics=("parallel",)),
    )(x, w_ih_t, w_hh_t, b, fc_w, fc_b)

    r_out = outs[0]
    model_out = list(outs[1:])
    return model_out, r_out


# speedup vs baseline: 1.4686x; 1.2738x over previous
import math

import jax
import jax.numpy as jnp
from jax.experimental import pallas as pl
from jax.experimental.pallas import tpu as pltpu

_NCORES = 1


def _tchunk(T):
    return math.gcd(T, 8)


def _fused_kernel(x_hbm, wih_ref, whh_ref, b_ref, fcw_ref, fcb_ref,
                  r_hbm, h0_hbm, h1_hbm, h2_hbm, h3_hbm,
                  xs_scr, gx_scr, rout_scr, head_scr, h_scr, c_scr,
                  in_sems, rout_sem, head_sem):
    T, Bc, I = xs_scr.shape
    Hp = h_scr.shape[1]
    FCp = fcb_ref.shape[1]
    C = h0_hbm.shape[2]
    j = pl.program_id(0)
    b0 = j * Bc
    TC = _tchunk(T)
    nchunks = T // TC
    head_hbm = (h0_hbm, h1_hbm, h2_hbm, h3_hbm)

    in_copies = []
    for t in range(T):
        cp = pltpu.make_async_copy(
            x_hbm.at[pl.ds(b0, Bc), t, :], xs_scr.at[t],
            in_sems.at[t // TC])
        cp.start()
        in_copies.append(cp)

    h_scr[...] = jnp.zeros_like(h_scr)
    c_scr[...] = jnp.zeros_like(c_scr)

    out_copies = []
    whh = whh_ref[...]
    fcw = fcw_ref[...]

    def project_chunk(ck):
        for t in range(ck * TC, (ck + 1) * TC):
            in_copies[t].wait()
        rows = ck * TC * Bc
        gx_scr[rows:rows + TC * Bc, :] = (
            jnp.dot(
                xs_scr[ck * TC:(ck + 1) * TC].reshape(TC * Bc, I),
                wih_ref[...], preferred_element_type=jnp.float32)
            + b_ref[...]
        )

    def recur_step(t):
        gates = gx_scr[t * Bc:(t + 1) * Bc, :] + jnp.dot(
            h_scr[...], whh, preferred_element_type=jnp.float32)
        i_g = jax.nn.sigmoid(gates[:, 0 * Hp:1 * Hp])
        f_g = jax.nn.sigmoid(gates[:, 1 * Hp:2 * Hp])
        g_g = jnp.tanh(gates[:, 2 * Hp:3 * Hp])
        o_g = jax.nn.sigmoid(gates[:, 3 * Hp:4 * Hp])
        c_new = f_g * c_scr[...] + i_g * g_g
        h_new = o_g * jnp.tanh(c_new)
        c_scr[...] = c_new
        h_scr[...] = h_new
        rout_scr[t] = h_new
        cp = pltpu.make_async_copy(
            rout_scr.at[t], r_hbm.at[pl.ds(b0, Bc), t, :], rout_sem.at[0])
        cp.start()
        out_copies.append(cp)

    def head_chunk(ck):
        head_scr[ck * TC:(ck + 1) * TC] = (
            jnp.dot(
                rout_scr[ck * TC:(ck + 1) * TC].reshape(TC * Bc, Hp),
                fcw, preferred_element_type=jnp.float32)
            + fcb_ref[...]
        ).reshape(TC, Bc, FCp)
        for t in range(ck * TC, (ck + 1) * TC):
            for f in range(FCp // C):
                cp = pltpu.make_async_copy(
                    head_scr.at[t, :, pl.ds(f * C, C)],
                    head_hbm[f].at[pl.ds(b0, Bc), t, :],
                    head_sem.at[0])
                cp.start()
                out_copies.append(cp)

    project_chunk(0)
    if nchunks > 1:
        project_chunk(1)
    for ck in range(nchunks):
        for t in range(ck * TC, (ck + 1) * TC):
            recur_step(t)
        if ck + 2 < nchunks:
            project_chunk(ck + 2)
        if ck >= 1:
            head_chunk(ck - 1)
    head_chunk(nchunks - 1)

    for cp in out_copies:
        cp.wait()


def kernel(x, w_ih_t, w_hh_t, b, fc_w, fc_b):
    B, T, I = x.shape
    Hp4 = w_ih_t.shape[1]
    Hp = Hp4 // 4
    FCp = fc_w.shape[1]
    F, C = 4, 128
    Bc = B // _NCORES
    TC = _tchunk(T)

    outs = pl.pallas_call(
        _fused_kernel,
        out_shape=(
            jax.ShapeDtypeStruct((B, T, Hp), jnp.float32),
        ) + tuple(
            jax.ShapeDtypeStruct((B, T, C), jnp.float32) for _ in range(F)
        ),
        grid=(_NCORES,),
        in_specs=[
            pl.BlockSpec(memory_space=pl.ANY),
            pl.BlockSpec((I, Hp4), lambda j: (0, 0)),
            pl.BlockSpec((Hp, Hp4), lambda j: (0, 0)),
            pl.BlockSpec((1, Hp4), lambda j: (0, 0)),
            pl.BlockSpec((Hp, FCp), lambda j: (0, 0)),
            pl.BlockSpec((1, FCp), lambda j: (0, 0)),
        ],
        out_specs=tuple(
            pl.BlockSpec(memory_space=pl.ANY) for _ in range(1 + F)),
        scratch_shapes=[
            pltpu.VMEM((T, Bc, I), jnp.float32),
            pltpu.VMEM((T * Bc, Hp4), jnp.float32),
            pltpu.VMEM((T, Bc, Hp), jnp.float32),
            pltpu.VMEM((T, Bc, FCp), jnp.float32),
            pltpu.VMEM((Bc, Hp), jnp.float32),
            pltpu.VMEM((Bc, Hp), jnp.float32),
            pltpu.SemaphoreType.DMA((T // TC,)),
            pltpu.SemaphoreType.DMA((1,)),
            pltpu.SemaphoreType.DMA((1,)),
        ],
        compiler_params=pltpu.CompilerParams(
            dimension_semantics=("parallel",)),
    )(x, w_ih_t, w_hh_t, b, fc_w, fc_b)

    r_out = outs[0]
    model_out = list(outs[1:])
    return model_out, r_out
